# Initial kernel scaffold; baseline (speedup 1.0000x reference)
import jax
import jax.numpy as jnp
from jax.experimental import pallas as pl


def kernel(attr, uv, face_indices, bake_resolution, device):
    raise NotImplementedError("write your pallas kernel here")


# speedup vs baseline: 1.3274x; 1.3274x over previous
import functools

import jax
import jax.numpy as jnp
from jax import lax
from jax.experimental import pallas as pl
from jax.experimental.pallas import tpu as pltpu
from jax.experimental.pallas import tpu_sc as plsc

RES = 256
P = RES * RES
PB = 256


def _sc_gather(table, idx):
    B = idx.shape[0]
    D = table.shape[1]
    n_workers = 32
    bpw = B // n_workers
    n_chunks = 2
    cw = bpw // n_chunks
    mesh = plsc.VectorSubcoreMesh(core_axis_name="c", subcore_axis_name="s")

    @functools.partial(
        pl.kernel,
        mesh=mesh,
        out_type=jax.ShapeDtypeStruct((B, D), jnp.float32),
        scratch_types=[
            pltpu.VMEM((n_chunks, cw), jnp.int32),
            pltpu.VMEM((cw, D), jnp.float32),
            pltpu.SemaphoreType.DMA,
        ],
    )
    def k(table_hbm, idx_hbm, out_hbm, idx_v, rows_v, sem):
        wid = lax.axis_index("s") * 2 + lax.axis_index("c")
        base = wid * bpw
        for j in range(n_chunks):
            pltpu.sync_copy(idx_hbm.at[pl.ds(base + j * cw, cw)], idx_v.at[j])
            pltpu.async_copy(table_hbm.at[idx_v.at[j]], rows_v, sem).wait()
            pltpu.sync_copy(rows_v, out_hbm.at[pl.ds(base + j * cw, cw)])

    return k(table, idx)


def _raster_body(vdata_ref, a0_ref, a1_ref, a2_ref, out_ref):
    F = a0_ref.shape[0]
    v0x = vdata_ref[0, 0:1, :]
    v0y = vdata_ref[0, 1:2, :]
    v1x = vdata_ref[1, 0:1, :]
    v1y = vdata_ref[1, 1:2, :]
    v2x = vdata_ref[2, 0:1, :]
    v2y = vdata_ref[2, 1:2, :]

    e0 = v1y - v2y
    e1 = v2x - v1x
    e2 = v2y - v0y
    e3 = v0x - v2x
    d = e0 * (v0x - v2x) + e1 * (v0y - v2y)
    valid = jnp.abs(d) > 1e-8
    d_safe = jnp.where(valid, d, 1.0)

    pid = pl.program_id(0)
    p = pid * PB + lax.broadcasted_iota(jnp.int32, (PB, 1), 0)
    pxs = ((p & (RES - 1)).astype(jnp.float32) + 0.5) / float(RES)
    pys = ((p >> 8).astype(jnp.float32) + 0.5) / float(RES)

    t0 = pxs - v2x
    t1 = pys - v2y
    u = (e0 * t0 + e1 * t1) / d_safe
    v = (e2 * t0 + e3 * t1) / d_safe
    w = 1.0 - u - v
    m = valid & (u >= 0.0) & (v >= 0.0) & (w >= 0.0)

    cols = lax.broadcasted_iota(jnp.int32, (PB, F), 1)
    fidx = jnp.min(jnp.where(m, cols, jnp.int32(1 << 30)), axis=1, keepdims=True)
    oh = cols == fidx
    m0 = jnp.where(oh, u, 0.0)
    m1 = jnp.where(oh, v, 0.0)
    m2 = jnp.where(oh, w, 0.0)

    acc = lax.dot(m0, a0_ref[...], precision=lax.Precision.HIGHEST)
    acc += lax.dot(m1, a1_ref[...], precision=lax.Precision.HIGHEST)
    acc += lax.dot(m2, a2_ref[...], precision=lax.Precision.HIGHEST)
    out_ref[...] = acc


def _bake(vdata, a0, a1, a2, interpret=False):
    F = a0.shape[0]
    return pl.pallas_call(
        _raster_body,
        grid=(P // PB,),
        in_specs=[
            pl.BlockSpec((3, 8, F), lambda i: (0, 0, 0)),
            pl.BlockSpec((F, 3), lambda i: (0, 0)),
            pl.BlockSpec((F, 3), lambda i: (0, 0)),
            pl.BlockSpec((F, 3), lambda i: (0, 0)),
        ],
        out_specs=pl.BlockSpec((PB, 3), lambda i: (i, 0)),
        out_shape=jax.ShapeDtypeStruct((P, 3), jnp.float32),
        compiler_params=pltpu.CompilerParams(
            dimension_semantics=("parallel",),
        ),
        interpret=interpret,
    )(vdata, a0, a1, a2)


def kernel(attr, uv, face_indices, bake_resolution, device):
    V = uv.shape[0]
    F = face_indices.shape[0]
    table = jnp.concatenate(
        [uv, attr, jnp.zeros((V, 123), jnp.float32)], axis=1
    )
    idx = face_indices.astype(jnp.int32).T.reshape(-1)
    g = _sc_gather(table, idx).reshape(3, F, 128)
    vdata = jnp.transpose(g[:, :, :8], (0, 2, 1))
    a0 = g[0, :, 2:5]
    a1 = g[1, :, 2:5]
    a2 = g[2, :, 2:5]
    out = _bake(vdata, a0, a1, a2)
    return out.reshape(RES, RES, 3)
